# Initial kernel scaffold; baseline (speedup 1.0000x reference)
import jax
import jax.numpy as jnp
from jax.experimental import pallas as pl


def kernel(x, edge_index, W, b):
    raise NotImplementedError("write your pallas kernel here")


# speedup vs baseline: 5.9210x; 5.9210x over previous
import functools

import jax
import jax.numpy as jnp
from jax import lax
from jax.experimental import pallas as pl
from jax.experimental.pallas import tpu as pltpu
from jax.experimental.pallas import tpu_sc as plsc

N_NODES = 10000
N_EDGES = 320000
DIM = 128

NC = 2
NS = 16
NW = NC * NS

CH = 128
G = 4
EPT = 10240
NGRP = EPT // (G * CH)
E_PAD = NW * EPT
ROWS = 10240
RPT = ROWS // NS
NBLK = RPT // CH


_sc_mesh = plsc.VectorSubcoreMesh(core_axis_name="c", subcore_axis_name="s")


@functools.partial(
    pl.kernel,
    mesh=_sc_mesh,
    out_type=jax.ShapeDtypeStruct((NC * ROWS, DIM), jnp.float32),
    scratch_types=[
        pltpu.VMEM((CH,), jnp.int32),
        pltpu.VMEM((G, CH), jnp.int32),
        pltpu.VMEM((CH, DIM), jnp.float32),
        pltpu.VMEM_SHARED((ROWS, DIM), jnp.float32),
        pltpu.SemaphoreType.DMA,
    ],
)
def _sc_sums(x_hbm, src_hbm, dst2_hbm, zrow_hbm,
             s_out, src_v, dst2_v, rows_v, acc, sem):
    cid = lax.axis_index("c")
    sid = lax.axis_index("s")
    wid = cid * NS + sid

    pltpu.sync_copy(zrow_hbm, rows_v)
    row0 = pl.multiple_of(sid * RPT, RPT)
    for j in range(NBLK):
        r0 = pl.multiple_of(row0 + j * CH, CH)
        pltpu.sync_copy(rows_v, acc.at[pl.ds(r0, CH)])
    plsc.subcore_barrier()

    ebase = wid * EPT
    gbase = wid * (EPT // CH)

    def body(g, carry):
        grow = pl.multiple_of(gbase + g * G, G)
        pltpu.sync_copy(dst2_hbm.at[pl.ds(grow, G)], dst2_v)
        for j in range(G):
            off = pl.multiple_of(ebase + g * (G * CH) + j * CH, CH)
            pltpu.sync_copy(src_hbm.at[pl.ds(off, CH)], src_v)
            pltpu.async_copy(x_hbm.at[src_v], rows_v, sem).wait()
            pltpu.sync_copy(rows_v, acc.at[dst2_v.at[j]], add=True)
        return carry

    lax.fori_loop(0, NGRP, body, 0)
    plsc.subcore_barrier()

    wb0 = cid * ROWS + sid * RPT
    for j in range(NBLK):
        r0 = pl.multiple_of(row0 + j * CH, CH)
        w0 = pl.multiple_of(wb0 + j * CH, CH)
        pltpu.sync_copy(acc.at[pl.ds(r0, CH)], rows_v)
        pltpu.sync_copy(rows_v, s_out.at[pl.ds(w0, CH)])


@functools.partial(
    pl.kernel,
    mesh=_sc_mesh,
    out_type=jax.ShapeDtypeStruct((NC * ROWS, DIM), jnp.float32),
    scratch_types=[
        pltpu.VMEM((G, CH), jnp.int32),
        pltpu.VMEM((CH, DIM), jnp.float32),
        pltpu.VMEM_SHARED((ROWS, DIM), jnp.float32),
    ],
)
def _sc_degree(dst2_hbm, zrow_hbm, ones_hbm,
               d_out, dst2_v, blk_v, dacc):
    cid = lax.axis_index("c")
    sid = lax.axis_index("s")
    wid = cid * NS + sid

    pltpu.sync_copy(zrow_hbm, blk_v)
    row0 = pl.multiple_of(sid * RPT, RPT)
    for j in range(NBLK):
        r0 = pl.multiple_of(row0 + j * CH, CH)
        pltpu.sync_copy(blk_v, dacc.at[pl.ds(r0, CH)])
    pltpu.sync_copy(ones_hbm, blk_v)
    plsc.subcore_barrier()

    gbase = wid * (EPT // CH)

    def body(g, carry):
        grow = pl.multiple_of(gbase + g * G, G)
        pltpu.sync_copy(dst2_hbm.at[pl.ds(grow, G)], dst2_v)
        for j in range(G):
            pltpu.sync_copy(blk_v, dacc.at[dst2_v.at[j]], add=True)
        return carry

    lax.fori_loop(0, NGRP, body, 0)
    plsc.subcore_barrier()

    wb0 = cid * ROWS + sid * RPT
    for j in range(NBLK):
        r0 = pl.multiple_of(row0 + j * CH, CH)
        w0 = pl.multiple_of(wb0 + j * CH, CH)
        pltpu.sync_copy(dacc.at[pl.ds(r0, CH)], blk_v)
        pltpu.sync_copy(blk_v, d_out.at[pl.ds(w0, CH)])


_R = 1024


def _tc_body(s_ref, d_ref, w_ref, b_ref, o_ref):
    s = s_ref[0] + s_ref[1]
    deg = d_ref[0, :, 0] + d_ref[1, :, 0]
    inv = 1.0 / jnp.maximum(deg, 1.0)
    mm = lax.dot_general(
        s, w_ref[...], (((1,), (1,)), ((), ())),
        precision=lax.Precision.HIGHEST,
        preferred_element_type=jnp.float32,
    )
    bias = jnp.where(deg > 0.0, 1.0, 0.0)[:, None] * b_ref[0][None, :]
    o_ref[...] = mm * inv[:, None] + bias


_tc_finalize = pl.pallas_call(
    _tc_body,
    grid=(ROWS // _R,),
    in_specs=[
        pl.BlockSpec((NC, _R, DIM), lambda i: (0, i, 0)),
        pl.BlockSpec((NC, _R, DIM), lambda i: (0, i, 0)),
        pl.BlockSpec((DIM, DIM), lambda i: (0, 0)),
        pl.BlockSpec((1, DIM), lambda i: (0, 0)),
    ],
    out_specs=pl.BlockSpec((_R, DIM), lambda i: (i, 0)),
    out_shape=jax.ShapeDtypeStruct((ROWS, DIM), jnp.float32),
)


def kernel(x, edge_index, W, b):
    src = edge_index[0]
    dst = edge_index[1]
    pad = E_PAD - N_EDGES
    pad_iota = jnp.arange(pad, dtype=jnp.int32)
    pad_src = pad_iota % N_NODES
    pad_dst = N_NODES + (pad_iota % (ROWS - N_NODES))
    src_p = jnp.concatenate([src, pad_src])
    dst_p = jnp.concatenate([dst, pad_dst]).reshape(E_PAD // CH, CH)
    zrow = jnp.zeros((CH, DIM), jnp.float32)
    ones = jnp.ones((CH, DIM), jnp.float32)

    s_part = _sc_sums(x, src_p, dst_p, zrow)
    d_part = _sc_degree(dst_p, zrow, ones)
    out = _tc_finalize(s_part.reshape(NC, ROWS, DIM),
                       d_part.reshape(NC, ROWS, DIM),
                       W, b.reshape(1, DIM))
    return out[:N_NODES]


# speedup vs baseline: 6.3919x; 1.0795x over previous
import functools

import jax
import jax.numpy as jnp
from jax import lax
from jax.experimental import pallas as pl
from jax.experimental.pallas import tpu as pltpu
from jax.experimental.pallas import tpu_sc as plsc

N_NODES = 10000
N_EDGES = 320000
DIM = 128

NC = 2
NS = 16
NW = NC * NS

CHP = 112
NCH = 92
EPT = CHP * NCH
NPAIR = NCH // 2
E_PAD = NW * EPT
E_ALLOC = E_PAD + 2 * CHP
DGRP = 4
ROWS = 10240
RPT = ROWS // NS
ZB = 80
NZB = RPT // ZB


_sc_mesh = plsc.VectorSubcoreMesh(core_axis_name="c", subcore_axis_name="s")


@functools.partial(
    pl.kernel,
    mesh=_sc_mesh,
    out_type=jax.ShapeDtypeStruct((NC * ROWS, DIM), jnp.float32),
    scratch_types=[
        pltpu.VMEM((CHP,), jnp.int32),
        pltpu.VMEM((CHP,), jnp.int32),
        pltpu.VMEM((2, CHP), jnp.int32),
        pltpu.VMEM((CHP, DIM), jnp.float32),
        pltpu.VMEM((CHP, DIM), jnp.float32),
        pltpu.VMEM_SHARED((ROWS, DIM), jnp.float32),
        pltpu.SemaphoreType.DMA,
        pltpu.SemaphoreType.DMA,
    ],
)
def _sc_sums(x_hbm, src_hbm, dst_hbm, zrow_hbm,
             s_out, src0_v, src1_v, dst2_v, rows0_v, rows1_v,
             acc, sem0, sem1):
    cid = lax.axis_index("c")
    sid = lax.axis_index("s")
    wid = cid * NS + sid

    pltpu.sync_copy(zrow_hbm, rows0_v)
    row0 = pl.multiple_of(sid * RPT, RPT)
    for j in range(NZB):
        r0 = pl.multiple_of(row0 + j * ZB, ZB)
        pltpu.sync_copy(rows0_v.at[pl.ds(0, ZB)], acc.at[pl.ds(r0, ZB)])
    plsc.subcore_barrier()

    ebase = wid * EPT

    e00 = pl.multiple_of(ebase, CHP)
    pltpu.sync_copy(dst_hbm.at[pl.ds(e00, CHP)], dst2_v.at[0])
    pltpu.sync_copy(dst_hbm.at[pl.ds(pl.multiple_of(ebase + CHP, CHP), CHP)],
                    dst2_v.at[1])
    pltpu.sync_copy(src_hbm.at[pl.ds(e00, CHP)], src0_v)
    pltpu.async_copy(x_hbm.at[src0_v], rows0_v, sem0)

    def body(t, carry):
        e0 = ebase + t * (2 * CHP)
        off1 = pl.multiple_of(e0 + CHP, CHP)
        pltpu.sync_copy(src_hbm.at[pl.ds(off1, CHP)], src1_v)
        pltpu.async_copy(x_hbm.at[src1_v], rows1_v, sem1)
        pltpu.make_async_copy(x_hbm.at[src0_v], rows0_v, sem0).wait()
        pltpu.sync_copy(rows0_v, acc.at[dst2_v.at[0]], add=True)
        off2 = pl.multiple_of(e0 + 2 * CHP, CHP)
        pltpu.sync_copy(src_hbm.at[pl.ds(off2, CHP)], src0_v)
        pltpu.async_copy(x_hbm.at[src0_v], rows0_v, sem0)
        pltpu.make_async_copy(x_hbm.at[src1_v], rows1_v, sem1).wait()
        pltpu.sync_copy(rows1_v, acc.at[dst2_v.at[1]], add=True)
        pltpu.sync_copy(dst_hbm.at[pl.ds(off2, CHP)], dst2_v.at[0])
        off3 = pl.multiple_of(e0 + 3 * CHP, CHP)
        pltpu.sync_copy(dst_hbm.at[pl.ds(off3, CHP)], dst2_v.at[1])
        return carry

    lax.fori_loop(0, NPAIR, body, 0)
    pltpu.make_async_copy(x_hbm.at[src0_v], rows0_v, sem0).wait()
    plsc.subcore_barrier()

    wb0 = cid * ROWS + sid * RPT
    for j in range(NZB):
        r0 = pl.multiple_of(row0 + j * ZB, ZB)
        w0 = pl.multiple_of(wb0 + j * ZB, ZB)
        pltpu.sync_copy(acc.at[pl.ds(r0, ZB)], rows0_v.at[pl.ds(0, ZB)])
        pltpu.sync_copy(rows0_v.at[pl.ds(0, ZB)], s_out.at[pl.ds(w0, ZB)])


@functools.partial(
    pl.kernel,
    mesh=_sc_mesh,
    out_type=jax.ShapeDtypeStruct((NC * ROWS, DIM), jnp.float32),
    scratch_types=[
        pltpu.VMEM((DGRP, CHP), jnp.int32),
        pltpu.VMEM((CHP, DIM), jnp.float32),
        pltpu.VMEM_SHARED((ROWS, DIM), jnp.float32),
    ],
)
def _sc_degree(dst_hbm, zrow_hbm, ones_hbm,
               d_out, dst2_v, blk_v, dacc):
    cid = lax.axis_index("c")
    sid = lax.axis_index("s")
    wid = cid * NS + sid

    pltpu.sync_copy(zrow_hbm, blk_v)
    row0 = pl.multiple_of(sid * RPT, RPT)
    for j in range(NZB):
        r0 = pl.multiple_of(row0 + j * ZB, ZB)
        pltpu.sync_copy(blk_v.at[pl.ds(0, ZB)], dacc.at[pl.ds(r0, ZB)])
    pltpu.sync_copy(ones_hbm, blk_v)
    plsc.subcore_barrier()

    ebase = wid * EPT

    def body(g, carry):
        for j in range(DGRP):
            off = pl.multiple_of(ebase + (g * DGRP + j) * CHP, CHP)
            pltpu.sync_copy(dst_hbm.at[pl.ds(off, CHP)], dst2_v.at[j])
        for j in range(DGRP):
            pltpu.sync_copy(blk_v, dacc.at[dst2_v.at[j]], add=True)
        return carry

    lax.fori_loop(0, NCH // DGRP, body, 0)
    plsc.subcore_barrier()

    wb0 = cid * ROWS + sid * RPT
    for j in range(NZB):
        r0 = pl.multiple_of(row0 + j * ZB, ZB)
        w0 = pl.multiple_of(wb0 + j * ZB, ZB)
        pltpu.sync_copy(dacc.at[pl.ds(r0, ZB)], blk_v.at[pl.ds(0, ZB)])
        pltpu.sync_copy(blk_v.at[pl.ds(0, ZB)], d_out.at[pl.ds(w0, ZB)])


_R = 1024


def _tc_body(s_ref, d_ref, w_ref, b_ref, o_ref):
    s = s_ref[0] + s_ref[1]
    deg = d_ref[0, :, 0] + d_ref[1, :, 0]
    inv = 1.0 / jnp.maximum(deg, 1.0)
    mm = lax.dot_general(
        s, w_ref[...], (((1,), (1,)), ((), ())),
        precision=lax.Precision.HIGHEST,
        preferred_element_type=jnp.float32,
    )
    bias = jnp.where(deg > 0.0, 1.0, 0.0)[:, None] * b_ref[0][None, :]
    o_ref[...] = mm * inv[:, None] + bias


_tc_finalize = pl.pallas_call(
    _tc_body,
    grid=(ROWS // _R,),
    in_specs=[
        pl.BlockSpec((NC, _R, DIM), lambda i: (0, i, 0)),
        pl.BlockSpec((NC, _R, DIM), lambda i: (0, i, 0)),
        pl.BlockSpec((DIM, DIM), lambda i: (0, 0)),
        pl.BlockSpec((1, DIM), lambda i: (0, 0)),
    ],
    out_specs=pl.BlockSpec((_R, DIM), lambda i: (i, 0)),
    out_shape=jax.ShapeDtypeStruct((ROWS, DIM), jnp.float32),
)


def kernel(x, edge_index, W, b):
    src = edge_index[0]
    dst = edge_index[1]
    pad = E_ALLOC - N_EDGES
    pad_iota = jnp.arange(pad, dtype=jnp.int32)
    pad_src = pad_iota % N_NODES
    pad_dst = N_NODES + (pad_iota % (ROWS - N_NODES))
    src_p = jnp.concatenate([src, pad_src])
    dst_p = jnp.concatenate([dst, pad_dst])
    zrow = jnp.zeros((CHP, DIM), jnp.float32)
    ones = jnp.ones((CHP, DIM), jnp.float32)

    s_part = _sc_sums(x, src_p, dst_p, zrow)
    d_part = _sc_degree(dst_p, zrow, ones)
    out = _tc_finalize(s_part.reshape(NC, ROWS, DIM),
                       d_part.reshape(NC, ROWS, DIM),
                       W, b.reshape(1, DIM))
    return out[:N_NODES]


# speedup vs baseline: 7.4623x; 1.1675x over previous
import functools

import jax
import jax.numpy as jnp
from jax import lax
from jax.experimental import pallas as pl
from jax.experimental.pallas import tpu as pltpu
from jax.experimental.pallas import tpu_sc as plsc

N_NODES = 10000
N_EDGES = 320000
DIM = 128

NC = 2
NS = 16
NW = NC * NS

CHP = 112
NCH = 92
EPT = CHP * NCH
NPAIR = NCH // 2
E_PAD = NW * EPT
E_ALLOC = E_PAD + 2 * CHP
DGRP = 4
CHD = 128
EPTD = 10240
E_DEG = NW * EPTD
ROWS = 10240
RPT = ROWS // NS
ZB = 80
NZB = RPT // ZB


_sc_mesh = plsc.VectorSubcoreMesh(core_axis_name="c", subcore_axis_name="s")


@functools.partial(
    pl.kernel,
    mesh=_sc_mesh,
    out_type=jax.ShapeDtypeStruct((NC * ROWS, DIM), jnp.float32),
    scratch_types=[
        pltpu.VMEM((2 * CHP,), jnp.int32),
        pltpu.VMEM((2 * CHP,), jnp.int32),
        pltpu.VMEM((2, CHP), jnp.int32),
        pltpu.VMEM((CHP, DIM), jnp.float32),
        pltpu.VMEM((CHP, DIM), jnp.float32),
        pltpu.VMEM_SHARED((ROWS, DIM), jnp.float32),
        pltpu.SemaphoreType.DMA,
        pltpu.SemaphoreType.DMA,
    ],
)
def _sc_sums(x_hbm, src_hbm, dst_hbm, zrow_hbm,
             s_out, srcA_v, srcB_v, dst2_v, rows0_v, rows1_v,
             acc, sem0, sem1):
    cid = lax.axis_index("c")
    sid = lax.axis_index("s")
    wid = cid * NS + sid

    pltpu.sync_copy(zrow_hbm.at[pl.ds(0, CHP)], rows0_v)
    row0 = pl.multiple_of(sid * RPT, RPT)
    for j in range(NZB):
        r0 = pl.multiple_of(row0 + j * ZB, ZB)
        pltpu.sync_copy(rows0_v.at[pl.ds(0, ZB)], acc.at[pl.ds(r0, ZB)])
    plsc.subcore_barrier()

    ebase = wid * EPT
    sA0 = srcA_v.at[pl.ds(0, CHP)]
    sA1 = srcA_v.at[pl.ds(CHP, CHP)]
    sB0 = srcB_v.at[pl.ds(0, CHP)]
    sB1 = srcB_v.at[pl.ds(CHP, CHP)]

    def load_dst_pair(e0):
        pltpu.sync_copy(dst_hbm.at[pl.ds(pl.multiple_of(e0, CHP), CHP)],
                        dst2_v.at[0])
        pltpu.sync_copy(dst_hbm.at[pl.ds(pl.multiple_of(e0 + CHP, CHP), CHP)],
                        dst2_v.at[1])

    pltpu.sync_copy(src_hbm.at[pl.ds(pl.multiple_of(ebase, 2 * CHP), 2 * CHP)],
                    srcA_v)
    load_dst_pair(ebase)
    pltpu.async_copy(x_hbm.at[sA0], rows0_v, sem0)

    def body(q, carry):
        e0 = ebase + q * (4 * CHP)
        pltpu.async_copy(x_hbm.at[sA1], rows1_v, sem1)
        pltpu.make_async_copy(x_hbm.at[sA0], rows0_v, sem0).wait()
        pltpu.sync_copy(rows0_v, acc.at[dst2_v.at[0]], add=True)
        pltpu.sync_copy(
            src_hbm.at[pl.ds(pl.multiple_of(e0 + 2 * CHP, 2 * CHP), 2 * CHP)],
            srcB_v)
        pltpu.async_copy(x_hbm.at[sB0], rows0_v, sem0)
        pltpu.make_async_copy(x_hbm.at[sA1], rows1_v, sem1).wait()
        pltpu.sync_copy(rows1_v, acc.at[dst2_v.at[1]], add=True)
        load_dst_pair(e0 + 2 * CHP)
        pltpu.async_copy(x_hbm.at[sB1], rows1_v, sem1)
        pltpu.make_async_copy(x_hbm.at[sB0], rows0_v, sem0).wait()
        pltpu.sync_copy(rows0_v, acc.at[dst2_v.at[0]], add=True)
        pltpu.sync_copy(
            src_hbm.at[pl.ds(pl.multiple_of(e0 + 4 * CHP, 2 * CHP), 2 * CHP)],
            srcA_v)
        pltpu.async_copy(x_hbm.at[sA0], rows0_v, sem0)
        pltpu.make_async_copy(x_hbm.at[sB1], rows1_v, sem1).wait()
        pltpu.sync_copy(rows1_v, acc.at[dst2_v.at[1]], add=True)
        load_dst_pair(e0 + 4 * CHP)
        return carry

    lax.fori_loop(0, NPAIR // 2, body, 0)
    pltpu.make_async_copy(x_hbm.at[sA0], rows0_v, sem0).wait()
    plsc.subcore_barrier()

    wb0 = cid * ROWS + sid * RPT
    for j in range(NZB):
        r0 = pl.multiple_of(row0 + j * ZB, ZB)
        w0 = pl.multiple_of(wb0 + j * ZB, ZB)
        pltpu.sync_copy(acc.at[pl.ds(r0, ZB)], rows0_v.at[pl.ds(0, ZB)])
        pltpu.sync_copy(rows0_v.at[pl.ds(0, ZB)], s_out.at[pl.ds(w0, ZB)])


@functools.partial(
    pl.kernel,
    mesh=_sc_mesh,
    out_type=jax.ShapeDtypeStruct((NC * ROWS, DIM), jnp.float32),
    scratch_types=[
        pltpu.VMEM((DGRP, CHD), jnp.int32),
        pltpu.VMEM((CHD, DIM), jnp.float32),
        pltpu.VMEM_SHARED((ROWS, DIM), jnp.float32),
    ],
)
def _sc_degree(dst2_hbm, zrow_hbm, ones_hbm,
               d_out, dst2_v, blk_v, dacc):
    cid = lax.axis_index("c")
    sid = lax.axis_index("s")
    wid = cid * NS + sid

    pltpu.sync_copy(zrow_hbm, blk_v)
    row0 = pl.multiple_of(sid * RPT, RPT)
    for j in range(NZB):
        r0 = pl.multiple_of(row0 + j * ZB, ZB)
        pltpu.sync_copy(blk_v.at[pl.ds(0, ZB)], dacc.at[pl.ds(r0, ZB)])
    pltpu.sync_copy(ones_hbm, blk_v)
    plsc.subcore_barrier()

    gbase = wid * (EPTD // CHD)

    def body(g, carry):
        grow = pl.multiple_of(gbase + g * DGRP, DGRP)
        pltpu.sync_copy(dst2_hbm.at[pl.ds(grow, DGRP)], dst2_v)
        for j in range(DGRP):
            pltpu.sync_copy(blk_v, dacc.at[dst2_v.at[j]], add=True)
        return carry

    lax.fori_loop(0, EPTD // (CHD * DGRP), body, 0)
    plsc.subcore_barrier()

    wb0 = cid * ROWS + sid * RPT
    for j in range(NZB):
        r0 = pl.multiple_of(row0 + j * ZB, ZB)
        w0 = pl.multiple_of(wb0 + j * ZB, ZB)
        pltpu.sync_copy(dacc.at[pl.ds(r0, ZB)], blk_v.at[pl.ds(0, ZB)])
        pltpu.sync_copy(blk_v.at[pl.ds(0, ZB)], d_out.at[pl.ds(w0, ZB)])


_R = 1024


def _tc_body(s_ref, d_ref, w_ref, b_ref, o_ref):
    s = s_ref[0] + s_ref[1]
    deg = d_ref[0, :, 0] + d_ref[1, :, 0]
    inv = 1.0 / jnp.maximum(deg, 1.0)
    mm = lax.dot_general(
        s, w_ref[...], (((1,), (1,)), ((), ())),
        precision=lax.Precision.HIGHEST,
        preferred_element_type=jnp.float32,
    )
    bias = jnp.where(deg > 0.0, 1.0, 0.0)[:, None] * b_ref[0][None, :]
    o_ref[...] = mm * inv[:, None] + bias


_tc_finalize = pl.pallas_call(
    _tc_body,
    grid=(ROWS // _R,),
    in_specs=[
        pl.BlockSpec((NC, _R, DIM), lambda i: (0, i, 0)),
        pl.BlockSpec((NC, _R, DIM), lambda i: (0, i, 0)),
        pl.BlockSpec((DIM, DIM), lambda i: (0, 0)),
        pl.BlockSpec((1, DIM), lambda i: (0, 0)),
    ],
    out_specs=pl.BlockSpec((_R, DIM), lambda i: (i, 0)),
    out_shape=jax.ShapeDtypeStruct((ROWS, DIM), jnp.float32),
)


def kernel(x, edge_index, W, b):
    src = edge_index[0]
    dst = edge_index[1]
    pad = E_ALLOC - N_EDGES
    pad_iota = jnp.arange(pad, dtype=jnp.int32)
    pad_src = pad_iota % N_NODES
    pad_dst = N_NODES + (pad_iota % (ROWS - N_NODES))
    src_p = jnp.concatenate([src, pad_src])
    dst_p = jnp.concatenate([dst, pad_dst])
    dst_2d = dst_p[:E_DEG].reshape(E_DEG // CHD, CHD)
    zrow = jnp.zeros((CHD, DIM), jnp.float32)
    ones = jnp.ones((CHD, DIM), jnp.float32)

    s_part = _sc_sums(x, src_p, dst_p, zrow)
    d_part = _sc_degree(dst_2d, zrow, ones)
    out = _tc_finalize(s_part.reshape(NC, ROWS, DIM),
                       d_part.reshape(NC, ROWS, DIM),
                       W, b.reshape(1, DIM))
    return out[:N_NODES]


# speedup vs baseline: 7.5083x; 1.0062x over previous
import functools

import jax
import jax.numpy as jnp
from jax import lax
from jax.experimental import pallas as pl
from jax.experimental.pallas import tpu as pltpu
from jax.experimental.pallas import tpu_sc as plsc

N_NODES = 10000
N_EDGES = 320000
DIM = 128

NC = 2
NS = 16
NW = NC * NS

CHP = 112
NCH = 92
EPT = CHP * NCH
NPAIR = NCH // 2
E_PAD = NW * EPT
E_ALLOC = E_PAD + 2 * CHP
DGRP = 4
CHD = 128
EPTD = 10240
E_DEG = NW * EPTD
ROWS = 10240
RPT = ROWS // NS
ZB = 80
NZB = RPT // ZB


_sc_mesh = plsc.VectorSubcoreMesh(core_axis_name="c", subcore_axis_name="s")


@functools.partial(
    pl.kernel,
    mesh=_sc_mesh,
    out_type=jax.ShapeDtypeStruct((NC * ROWS, DIM), jnp.float32),
    scratch_types=[
        pltpu.VMEM((2 * CHP,), jnp.int32),
        pltpu.VMEM((2 * CHP,), jnp.int32),
        pltpu.VMEM((2, CHP), jnp.int32),
        pltpu.VMEM((CHP, DIM), jnp.float32),
        pltpu.VMEM((CHP, DIM), jnp.float32),
        pltpu.VMEM_SHARED((ROWS, DIM), jnp.float32),
        pltpu.SemaphoreType.DMA,
        pltpu.SemaphoreType.DMA,
    ],
)
def _sc_sums(x_hbm, src_hbm, dst_hbm, zrow_hbm,
             s_out, srcA_v, srcB_v, dst2_v, rows0_v, rows1_v,
             acc, sem0, sem1):
    cid = lax.axis_index("c")
    sid = lax.axis_index("s")
    wid = cid * NS + sid

    pltpu.sync_copy(zrow_hbm.at[pl.ds(0, CHP)], rows0_v)
    row0 = pl.multiple_of(sid * RPT, RPT)
    for j in range(NZB):
        r0 = pl.multiple_of(row0 + j * ZB, ZB)
        pltpu.sync_copy(rows0_v.at[pl.ds(0, ZB)], acc.at[pl.ds(r0, ZB)])
    plsc.subcore_barrier()

    ebase = wid * EPT
    sA0 = srcA_v.at[pl.ds(0, CHP)]
    sA1 = srcA_v.at[pl.ds(CHP, CHP)]
    sB0 = srcB_v.at[pl.ds(0, CHP)]
    sB1 = srcB_v.at[pl.ds(CHP, CHP)]

    def load_dst_pair(e0):
        pltpu.sync_copy(dst_hbm.at[pl.ds(pl.multiple_of(e0, CHP), CHP)],
                        dst2_v.at[0])
        pltpu.sync_copy(dst_hbm.at[pl.ds(pl.multiple_of(e0 + CHP, CHP), CHP)],
                        dst2_v.at[1])

    pltpu.sync_copy(src_hbm.at[pl.ds(pl.multiple_of(ebase, 2 * CHP), 2 * CHP)],
                    srcA_v)
    load_dst_pair(ebase)
    pltpu.async_copy(x_hbm.at[sA0], rows0_v, sem0)

    def body(q, carry):
        e0 = ebase + q * (4 * CHP)
        pltpu.async_copy(x_hbm.at[sA1], rows1_v, sem1)
        pltpu.make_async_copy(x_hbm.at[sA0], rows0_v, sem0).wait()
        pltpu.sync_copy(rows0_v, acc.at[dst2_v.at[0]], add=True)
        pltpu.sync_copy(
            src_hbm.at[pl.ds(pl.multiple_of(e0 + 2 * CHP, 2 * CHP), 2 * CHP)],
            srcB_v)
        pltpu.async_copy(x_hbm.at[sB0], rows0_v, sem0)
        pltpu.make_async_copy(x_hbm.at[sA1], rows1_v, sem1).wait()
        pltpu.sync_copy(rows1_v, acc.at[dst2_v.at[1]], add=True)
        load_dst_pair(e0 + 2 * CHP)
        pltpu.async_copy(x_hbm.at[sB1], rows1_v, sem1)
        pltpu.make_async_copy(x_hbm.at[sB0], rows0_v, sem0).wait()
        pltpu.sync_copy(rows0_v, acc.at[dst2_v.at[0]], add=True)
        pltpu.sync_copy(
            src_hbm.at[pl.ds(pl.multiple_of(e0 + 4 * CHP, 2 * CHP), 2 * CHP)],
            srcA_v)
        pltpu.async_copy(x_hbm.at[sA0], rows0_v, sem0)
        pltpu.make_async_copy(x_hbm.at[sB1], rows1_v, sem1).wait()
        pltpu.sync_copy(rows1_v, acc.at[dst2_v.at[1]], add=True)
        load_dst_pair(e0 + 4 * CHP)
        return carry

    lax.fori_loop(0, NPAIR // 2, body, 0)
    pltpu.make_async_copy(x_hbm.at[sA0], rows0_v, sem0).wait()
    plsc.subcore_barrier()

    wb0 = cid * ROWS + sid * RPT
    for j in range(NZB):
        r0 = pl.multiple_of(row0 + j * ZB, ZB)
        w0 = pl.multiple_of(wb0 + j * ZB, ZB)
        pltpu.sync_copy(acc.at[pl.ds(r0, ZB)], rows0_v.at[pl.ds(0, ZB)])
        pltpu.sync_copy(rows0_v.at[pl.ds(0, ZB)], s_out.at[pl.ds(w0, ZB)])


@functools.partial(
    pl.kernel,
    mesh=_sc_mesh,
    out_type=jax.ShapeDtypeStruct((NC * ROWS, DIM), jnp.float32),
    scratch_types=[
        pltpu.VMEM((DGRP, CHD), jnp.int32),
        pltpu.VMEM((CHD, DIM), jnp.float32),
        pltpu.VMEM_SHARED((ROWS, DIM), jnp.float32),
    ],
)
def _sc_degree(dst2_hbm, zrow_hbm, ones_hbm,
               d_out, dst2_v, blk_v, dacc):
    cid = lax.axis_index("c")
    sid = lax.axis_index("s")
    wid = cid * NS + sid

    pltpu.sync_copy(zrow_hbm, blk_v)
    row0 = pl.multiple_of(sid * RPT, RPT)
    for j in range(NZB):
        r0 = pl.multiple_of(row0 + j * ZB, ZB)
        pltpu.sync_copy(blk_v.at[pl.ds(0, ZB)], dacc.at[pl.ds(r0, ZB)])
    pltpu.sync_copy(ones_hbm, blk_v)
    plsc.subcore_barrier()

    gbase = wid * (EPTD // CHD)

    def body(g, carry):
        grow = pl.multiple_of(gbase + g * DGRP, DGRP)
        pltpu.sync_copy(dst2_hbm.at[pl.ds(grow, DGRP)], dst2_v)
        for j in range(DGRP):
            pltpu.sync_copy(blk_v, dacc.at[dst2_v.at[j]], add=True)
        return carry

    lax.fori_loop(0, EPTD // (CHD * DGRP), body, 0)
    plsc.subcore_barrier()

    wb0 = cid * ROWS + sid * RPT
    for j in range(NZB):
        r0 = pl.multiple_of(row0 + j * ZB, ZB)
        w0 = pl.multiple_of(wb0 + j * ZB, ZB)
        pltpu.sync_copy(dacc.at[pl.ds(r0, ZB)], blk_v.at[pl.ds(0, ZB)])
        pltpu.sync_copy(blk_v.at[pl.ds(0, ZB)], d_out.at[pl.ds(w0, ZB)])


_R = 1024


def _tc_body(s_ref, d_ref, w_ref, b_ref, o_ref):
    s = s_ref[0] + s_ref[1]
    deg = d_ref[0, :, 0] + d_ref[1, :, 0]
    inv = 1.0 / jnp.maximum(deg, 1.0)
    mm = lax.dot_general(
        s, w_ref[...], (((1,), (1,)), ((), ())),
        preferred_element_type=jnp.float32,
    )
    bias = jnp.where(deg > 0.0, 1.0, 0.0)[:, None] * b_ref[0][None, :]
    o_ref[...] = mm * inv[:, None] + bias


_tc_finalize = pl.pallas_call(
    _tc_body,
    grid=(ROWS // _R,),
    in_specs=[
        pl.BlockSpec((NC, _R, DIM), lambda i: (0, i, 0)),
        pl.BlockSpec((NC, _R, DIM), lambda i: (0, i, 0)),
        pl.BlockSpec((DIM, DIM), lambda i: (0, 0)),
        pl.BlockSpec((1, DIM), lambda i: (0, 0)),
    ],
    out_specs=pl.BlockSpec((_R, DIM), lambda i: (i, 0)),
    out_shape=jax.ShapeDtypeStruct((ROWS, DIM), jnp.float32),
)


def kernel(x, edge_index, W, b):
    src = edge_index[0]
    dst = edge_index[1]
    pad = E_ALLOC - N_EDGES
    pad_iota = jnp.arange(pad, dtype=jnp.int32)
    pad_src = pad_iota % N_NODES
    pad_dst = N_NODES + (pad_iota % (ROWS - N_NODES))
    src_p = jnp.concatenate([src, pad_src])
    dst_p = jnp.concatenate([dst, pad_dst])
    dst_2d = dst_p[:E_DEG].reshape(E_DEG // CHD, CHD)
    zrow = jnp.zeros((CHD, DIM), jnp.float32)
    ones = jnp.ones((CHD, DIM), jnp.float32)

    s_part = _sc_sums(x, src_p, dst_p, zrow)
    d_part = _sc_degree(dst_2d, zrow, ones)
    out = _tc_finalize(s_part.reshape(NC, ROWS, DIM),
                       d_part.reshape(NC, ROWS, DIM),
                       W, b.reshape(1, DIM))
    return out[:N_NODES]
